# baseline (speedup 1.0000x reference)
import functools

import jax
import jax.numpy as jnp
import numpy as np
from jax import lax
from jax.experimental import pallas as pl
from jax.experimental.pallas import tpu as pltpu
from jax.experimental.pallas import tpu_sc as plsc

_ALPHA = 0.5
_SAMPLE_KEY = 123
_B = 128
_N = 4096
_LANES = 16
_NC = 2
_NS = 16
_NW = _NC * _NS
_BATCHES_PER_TILE = _B // _NW

def _rotl32(x: np.ndarray, r: int) -> np.ndarray:
    return (x << np.uint32(r)) | (x >> np.uint32(32 - r))


def _sorted_uniforms() -> np.ndarray:
    n = _B * _N
    seed = _SAMPLE_KEY
    ks0 = np.uint32((seed >> 32) & 0xFFFFFFFF)
    ks1 = np.uint32(seed & 0xFFFFFFFF)
    ks2 = np.uint32(ks0 ^ ks1 ^ np.uint32(0x1BD11BDA))
    x0 = np.zeros(n, dtype=np.uint32)
    x1 = np.arange(n, dtype=np.uint32)
    rot_a = (13, 15, 26, 6)
    rot_b = (17, 29, 16, 24)
    with np.errstate(over="ignore"):
        x0 += ks0
        x1 += ks1
        for i, (ka, kb) in enumerate(
                [(ks1, ks2), (ks2, ks0), (ks0, ks1), (ks1, ks2), (ks2, ks0)]):
            for r in (rot_a if i % 2 == 0 else rot_b):
                x0 += x1
                x1 = _rotl32(x1, r)
                x1 ^= x0
            x0 += ka
            x1 += kb + np.uint32(i + 1)
    bits = x0 ^ x1
    fl = ((bits >> np.uint32(9)) | np.uint32(0x3F800000)).view(np.float32)
    u = (fl - np.float32(1.0)).reshape(_B, _N)
    return np.sort(u, axis=-1)


_U_SORTED = _sorted_uniforms()

_G_COLS = _N + 16


def _bucket_table() -> tuple[np.ndarray, list[int]]:
    grid = np.arange(_N + 2, dtype=np.float64) / float(_N)
    g = np.stack([np.searchsorted(_U_SORTED[b], grid, side="right")
                  for b in range(_B)]).astype(np.int32)
    gap = int((g[:, 1:] - g[:, :-1]).max())
    s = 1
    while 2 ** s - 1 < gap:
        s += 1
    halvings = [2 ** i for i in range(s - 1, -1, -1)]
    gp = np.full((_B, _G_COLS), _N, dtype=np.int32)
    gp[:, : _N + 2] = g
    return gp, halvings


_G_TABLE, _HALVINGS = _bucket_table()


def _resample_body(cdf_hbm, pw_hbm, ps_hbm, u_hbm, g_hbm, ops_hbm, opw_hbm,
                   *scratch):
    (v_cdf0, v_u0, v_g0, v_pw0, v_ps0, v_psx0, v_pwx0,
     v_cdf1, v_u1, v_g1, v_pw1, v_ps1, v_psx1, v_pwx1,
     v_f, v_ops0, v_opw0, v_ops1, v_opw1,
     sin0, sin1, sout0, sout1) = scratch
    ins = [(v_cdf0, v_u0, v_g0, v_pw0, v_ps0, v_psx0, v_pwx0, sin0),
           (v_cdf1, v_u1, v_g1, v_pw1, v_ps1, v_psx1, v_pwx1, sin1)]
    outs = [(v_ops0, v_opw0, sout0), (v_ops1, v_opw1, sout1)]

    wid = lax.axis_index("s") * _NC + lax.axis_index("c")
    lane = lax.iota(jnp.int32, _LANES)

    def issue_in(t, slot):
        v_cdf, v_u, v_g, v_pw, v_ps, v_psx, v_pwx, sem = ins[slot]
        b = wid * _BATCHES_PER_TILE + t
        bn = jnp.minimum(b + 1, _B - 1)
        return [
            pltpu.async_copy(cdf_hbm.at[b], v_cdf, sem),
            pltpu.async_copy(u_hbm.at[b], v_u, sem),
            pltpu.async_copy(g_hbm.at[b], v_g, sem),
            pltpu.async_copy(pw_hbm.at[b], v_pw, sem),
            pltpu.async_copy(ps_hbm.at[b], v_ps, sem),
            pltpu.async_copy(ps_hbm.at[bn, pl.ds(0, _LANES * 3)], v_psx, sem),
            pltpu.async_copy(pw_hbm.at[bn, pl.ds(0, _LANES)], v_pwx, sem),
        ]

    pending_in = issue_in(0, 0)
    pending_out = [None, None]
    for t in range(_BATCHES_PER_TILE):
        slot = t % 2
        b = wid * _BATCHES_PER_TILE + t
        nxt = (issue_in(t + 1, 1 - slot)
               if t + 1 < _BATCHES_PER_TILE else None)
        for d in pending_in:
            d.wait()
        pending_in = nxt
        v_cdf, v_u, v_g, v_pw, v_ps, v_psx, v_pwx, _ = ins[slot]
        v_ops, v_opw, sout = outs[slot]
        if pending_out[slot] is not None:
            for d in pending_out[slot]:
                d.wait()

        bx = b < _B - 1
        zero16 = jnp.zeros((_LANES,), jnp.int32)
        xval = [plsc.load_gather(v_psx, [jnp.full((_LANES,), k, jnp.int32)])
                for k in range(3)]
        xw = plsc.load_gather(v_pwx, [zero16])

        def _pass1(i, _):
            jbase = i * _LANES
            c = v_cdf[pl.ds(jbase, _LANES)]
            k = jnp.minimum((c * float(_N)).astype(jnp.int32), _N)
            pos = plsc.load_gather(v_g, [k])
            hi = plsc.load_gather(v_g, [k + 1])
            for h in _HALVINGS:
                probe = pos + h
                uval = plsc.load_gather(
                    v_u, [jnp.minimum(probe - 1, _N - 1)])
                ok = (probe <= hi) & (uval <= c)
                pos = pos + jnp.where(ok, h, 0)
            pos = jnp.where((i == _N // _LANES - 1) & (lane == _LANES - 1),
                            _N, pos)
            v_f[pl.ds(jbase, _LANES)] = pos
            return 0

        lax.fori_loop(0, _N // _LANES, _pass1, 0, unroll=4)

        def _pass2(i, _):
            jbase = i * _LANES
            jvec = lane + jbase
            f_cur = v_f[pl.ds(jbase, _LANES)]
            f_prev = plsc.load_gather(v_f, [jnp.maximum(jvec - 1, 0)])
            f_prev = jnp.where(jvec == 0, 0, f_prev)
            cnt = f_cur - f_prev
            cm = jnp.minimum(cnt, _N - 1)
            is_x = (cnt == _N) & bx
            cm3 = cm * 3
            jvec3 = jvec * 3
            for kk in range(3):
                val = plsc.load_gather(v_ps, [cm3 + kk])
                val = jnp.where(is_x, xval[kk], val)
                plsc.store_scatter(v_ops, [jvec3 + kk], val)
            wv = plsc.load_gather(v_pw, [cm])
            wv = jnp.where(is_x, xw, wv)
            v_opw[pl.ds(jbase, _LANES)] = wv
            return 0

        lax.fori_loop(0, _N // _LANES, _pass2, 0, unroll=4)

        pending_out[slot] = [
            pltpu.async_copy(v_ops, ops_hbm.at[b], sout),
            pltpu.async_copy(v_opw, opw_hbm.at[b], sout),
        ]
    for p in pending_out:
        if p is not None:
            for d in p:
                d.wait()


@functools.partial(
    pl.kernel,
    out_type=(
        jax.ShapeDtypeStruct((_B, _N * 3), jnp.float32),
        jax.ShapeDtypeStruct((_B, _N), jnp.float32),
    ),
    mesh=plsc.VectorSubcoreMesh(core_axis_name="c", subcore_axis_name="s"),
    compiler_params=pltpu.CompilerParams(
        needs_layout_passes=False, use_tc_tiling_on_sc=False),
    scratch_types=(
        [pltpu.VMEM((_N,), jnp.float32),
         pltpu.VMEM((_N,), jnp.float32),
         pltpu.VMEM((_G_COLS,), jnp.int32),
         pltpu.VMEM((_N,), jnp.float32),
         pltpu.VMEM((_N * 3,), jnp.float32),
         pltpu.VMEM((_LANES * 3,), jnp.float32),
         pltpu.VMEM((_LANES,), jnp.float32),
         ] * 2
        + [pltpu.VMEM((_N,), jnp.int32)]
        + [pltpu.VMEM((_N * 3,), jnp.float32),
           pltpu.VMEM((_N,), jnp.float32),
           ] * 2
        + [pltpu.SemaphoreType.DMA] * 4
    ),
)
def _resample_sc(cdf, pw_final, ps_rows, u_sorted, g_table, ops, opw,
                 *scratch):
    _resample_body(cdf, pw_final, ps_rows, u_sorted, g_table, ops, opw,
                   *scratch)


def kernel(particle_states, particle_weights):
    batch_size, num_particles, _ = particle_states.shape
    pw = particle_weights - jax.scipy.special.logsumexp(
        particle_weights, axis=-1, keepdims=True)
    uniform_weights = jnp.ones(
        (batch_size, num_particles), dtype=jnp.float32) * (-np.log(num_particles))
    q = jnp.stack([pw + np.log(_ALPHA), uniform_weights + np.log(1.0 - _ALPHA)],
                  axis=-1)
    q = jax.scipy.special.logsumexp(q, axis=-1)
    q = q - jax.scipy.special.logsumexp(q, axis=-1, keepdims=True)
    pw = pw - q
    probs = jax.nn.softmax(q, axis=-1)
    cdf = jnp.cumsum(probs, axis=-1)

    u_sorted = jnp.asarray(_U_SORTED)
    g_table = jnp.asarray(_G_TABLE)
    ps_rows = particle_states.reshape(batch_size, num_particles * 3)
    ps_out, pw_out = _resample_sc(cdf, pw, ps_rows, u_sorted, g_table)
    return (ps_out.reshape(batch_size, num_particles, 3), pw_out)


# speedup vs baseline: 1.6521x; 1.6521x over previous
import functools

import jax
import jax.numpy as jnp
import numpy as np
from jax import lax
from jax.experimental import pallas as pl
from jax.experimental.pallas import tpu as pltpu
from jax.experimental.pallas import tpu_sc as plsc

_ALPHA = 0.5
_SAMPLE_KEY = 123
_B = 128
_N = 4096
_LANES = 16
_NC = 2
_NS = 16
_NW = _NC * _NS
_BATCHES_PER_TILE = _B // _NW

def _rotl32(x: np.ndarray, r: int) -> np.ndarray:
    return (x << np.uint32(r)) | (x >> np.uint32(32 - r))


def _sorted_uniforms() -> np.ndarray:
    n = _B * _N
    seed = _SAMPLE_KEY
    ks0 = np.uint32((seed >> 32) & 0xFFFFFFFF)
    ks1 = np.uint32(seed & 0xFFFFFFFF)
    ks2 = np.uint32(ks0 ^ ks1 ^ np.uint32(0x1BD11BDA))
    x0 = np.zeros(n, dtype=np.uint32)
    x1 = np.arange(n, dtype=np.uint32)
    rot_a = (13, 15, 26, 6)
    rot_b = (17, 29, 16, 24)
    with np.errstate(over="ignore"):
        x0 += ks0
        x1 += ks1
        for i, (ka, kb) in enumerate(
                [(ks1, ks2), (ks2, ks0), (ks0, ks1), (ks1, ks2), (ks2, ks0)]):
            for r in (rot_a if i % 2 == 0 else rot_b):
                x0 += x1
                x1 = _rotl32(x1, r)
                x1 ^= x0
            x0 += ka
            x1 += kb + np.uint32(i + 1)
    bits = x0 ^ x1
    fl = ((bits >> np.uint32(9)) | np.uint32(0x3F800000)).view(np.float32)
    u = (fl - np.float32(1.0)).reshape(_B, _N)
    return np.sort(u, axis=-1)


_U_SORTED = _sorted_uniforms()

_G_COLS = _N + 16


def _bucket_table() -> tuple[np.ndarray, list[int]]:
    grid = np.arange(_N + 2, dtype=np.float64) / float(_N)
    g = np.stack([np.searchsorted(_U_SORTED[b], grid, side="right")
                  for b in range(_B)]).astype(np.int32)
    gap = int((g[:, 1:] - g[:, :-1]).max())
    s = 1
    while 2 ** s - 1 < gap:
        s += 1
    halvings = [2 ** i for i in range(s - 1, -1, -1)]
    gp = np.full((_B, _G_COLS), _N, dtype=np.int32)
    gp[:, : _N + 2] = g
    return gp, halvings


_G_TABLE, _HALVINGS = _bucket_table()


def _resample_body(cdf_hbm, pw_hbm, ps_hbm, u_hbm, g_hbm, ops_hbm, opw_hbm,
                   *scratch):
    (v_cdf0, v_u0, v_g0, v_pw0, v_ps0, v_psx0, v_pwx0,
     v_cdf1, v_u1, v_g1, v_pw1, v_ps1, v_psx1, v_pwx1,
     v_ops0, v_opw0, v_ops1, v_opw1,
     sin0, sin1, sout0, sout1) = scratch
    ins = [(v_cdf0, v_u0, v_g0, v_pw0, v_ps0, v_psx0, v_pwx0, sin0),
           (v_cdf1, v_u1, v_g1, v_pw1, v_ps1, v_psx1, v_pwx1, sin1)]
    outs = [(v_ops0, v_opw0, sout0), (v_ops1, v_opw1, sout1)]

    wid = lax.axis_index("s") * _NC + lax.axis_index("c")
    lane = lax.iota(jnp.int32, _LANES)

    def issue_in(t, slot):
        v_cdf, v_u, v_g, v_pw, v_ps, v_psx, v_pwx, sem = ins[slot]
        b = wid * _BATCHES_PER_TILE + t
        bn = jnp.minimum(b + 1, _B - 1)
        return [
            pltpu.async_copy(cdf_hbm.at[b], v_cdf, sem),
            pltpu.async_copy(u_hbm.at[b], v_u, sem),
            pltpu.async_copy(g_hbm.at[b], v_g, sem),
            pltpu.async_copy(pw_hbm.at[b], v_pw, sem),
            pltpu.async_copy(ps_hbm.at[b], v_ps, sem),
            pltpu.async_copy(ps_hbm.at[bn, pl.ds(0, _LANES * 3)], v_psx, sem),
            pltpu.async_copy(pw_hbm.at[bn, pl.ds(0, _LANES)], v_pwx, sem),
        ]

    pending_in = issue_in(0, 0)
    pending_out = [None, None]
    for t in range(_BATCHES_PER_TILE):
        slot = t % 2
        b = wid * _BATCHES_PER_TILE + t
        nxt = (issue_in(t + 1, 1 - slot)
               if t + 1 < _BATCHES_PER_TILE else None)
        for d in pending_in:
            d.wait()
        pending_in = nxt
        v_cdf, v_u, v_g, v_pw, v_ps, v_psx, v_pwx, _ = ins[slot]
        v_ops, v_opw, sout = outs[slot]
        if pending_out[slot] is not None:
            for d in pending_out[slot]:
                d.wait()

        bx = b < _B - 1
        zero16 = jnp.zeros((_LANES,), jnp.int32)
        xval = [plsc.load_gather(v_psx, [jnp.full((_LANES,), k, jnp.int32)])
                for k in range(3)]
        xw = plsc.load_gather(v_pwx, [zero16])

        def _search(c):
            k = jnp.minimum((c * float(_N)).astype(jnp.int32), _N)
            pos = plsc.load_gather(v_g, [k])
            hi = plsc.load_gather(v_g, [k + 1])
            for h in _HALVINGS:
                probe = pos + h
                uval = plsc.load_gather(
                    v_u, [jnp.minimum(probe - 1, _N - 1)])
                ok = (probe <= hi) & (uval <= c)
                pos = pos + jnp.where(ok, h, 0)
            return pos

        @functools.partial(plsc.parallel_loop, 0, _N // _LANES, unroll=4)
        def _pass(i):
            jbase = i * _LANES
            jvec = lane + jbase
            c = v_cdf[pl.ds(jbase, _LANES)]
            cp = plsc.load_gather(v_cdf, [jnp.maximum(jvec - 1, 0)])
            pos = _search(c)
            pos = jnp.where((i == _N // _LANES - 1) & (lane == _LANES - 1),
                            _N, pos)
            f_prev = jnp.where(jvec == 0, 0, _search(cp))
            cnt = pos - f_prev
            cm = jnp.minimum(cnt, _N - 1)
            is_x = (cnt == _N) & bx
            cm3 = cm * 3
            jvec3 = jvec * 3
            for kk in range(3):
                val = plsc.load_gather(v_ps, [cm3 + kk])
                val = jnp.where(is_x, xval[kk], val)
                plsc.store_scatter(v_ops, [jvec3 + kk], val)
            wv = plsc.load_gather(v_pw, [cm])
            wv = jnp.where(is_x, xw, wv)
            v_opw[pl.ds(jbase, _LANES)] = wv

        pending_out[slot] = [
            pltpu.async_copy(v_ops, ops_hbm.at[b], sout),
            pltpu.async_copy(v_opw, opw_hbm.at[b], sout),
        ]
    for p in pending_out:
        if p is not None:
            for d in p:
                d.wait()


@functools.partial(
    pl.kernel,
    out_type=(
        jax.ShapeDtypeStruct((_B, _N * 3), jnp.float32),
        jax.ShapeDtypeStruct((_B, _N), jnp.float32),
    ),
    mesh=plsc.VectorSubcoreMesh(core_axis_name="c", subcore_axis_name="s"),
    compiler_params=pltpu.CompilerParams(
        needs_layout_passes=False, use_tc_tiling_on_sc=False),
    scratch_types=(
        [pltpu.VMEM((_N,), jnp.float32),
         pltpu.VMEM((_N,), jnp.float32),
         pltpu.VMEM((_G_COLS,), jnp.int32),
         pltpu.VMEM((_N,), jnp.float32),
         pltpu.VMEM((_N * 3,), jnp.float32),
         pltpu.VMEM((_LANES * 3,), jnp.float32),
         pltpu.VMEM((_LANES,), jnp.float32),
         ] * 2
        + [pltpu.VMEM((_N * 3,), jnp.float32),
           pltpu.VMEM((_N,), jnp.float32),
           ] * 2
        + [pltpu.SemaphoreType.DMA] * 4
    ),
)
def _resample_sc(cdf, pw_final, ps_rows, u_sorted, g_table, ops, opw,
                 *scratch):
    _resample_body(cdf, pw_final, ps_rows, u_sorted, g_table, ops, opw,
                   *scratch)


def kernel(particle_states, particle_weights):
    batch_size, num_particles, _ = particle_states.shape
    pw = particle_weights - jax.scipy.special.logsumexp(
        particle_weights, axis=-1, keepdims=True)
    uniform_weights = jnp.ones(
        (batch_size, num_particles), dtype=jnp.float32) * (-np.log(num_particles))
    q = jnp.stack([pw + np.log(_ALPHA), uniform_weights + np.log(1.0 - _ALPHA)],
                  axis=-1)
    q = jax.scipy.special.logsumexp(q, axis=-1)
    q = q - jax.scipy.special.logsumexp(q, axis=-1, keepdims=True)
    pw = pw - q
    probs = jax.nn.softmax(q, axis=-1)
    cdf = jnp.cumsum(probs, axis=-1)

    u_sorted = jnp.asarray(_U_SORTED)
    g_table = jnp.asarray(_G_TABLE)
    ps_rows = particle_states.reshape(batch_size, num_particles * 3)
    ps_out, pw_out = _resample_sc(cdf, pw, ps_rows, u_sorted, g_table)
    return (ps_out.reshape(batch_size, num_particles, 3), pw_out)
